# baseline (speedup 1.0000x reference)
import functools

import jax
import jax.numpy as jnp
from jax import lax
from jax.experimental import pallas as pl
from jax.experimental.pallas import tpu as pltpu
from jax.experimental.pallas import tpu_sc as plsc

N = 10000
E = 320000
F = 128
K = 2
NH = 3

NC = 2
NS = 16
NW = NC * NS

CHUNK = 128
RPT = 80
R_PAD = RPT * NW
E_PAD = R_PAD * CHUNK
NBUF = 2
RPT0 = 56
RPT1 = 104
RPT_MAX = max(RPT0, RPT1)
N_PAD = 10240
ROWS_PER_TILE = N_PAD // NS


def _sc_mesh():
    return plsc.VectorSubcoreMesh(core_axis_name="c", subcore_axis_name="s")


def _deg_body(packed3, deg_out, pk_v, idx_d, ones_v, stripe_v, deg_acc, sem):
    c = lax.axis_index("c")
    s = lax.axis_index("s")
    wid = c * NS + s

    ones16 = jnp.ones((16,), jnp.float32)
    zeros16 = jnp.zeros((16,), jnp.float32)
    for j in range(CHUNK // 16):
        ones_v[pl.ds(j * 16, 16)] = ones16

    pltpu.async_copy(packed3.at[wid], pk_v, sem).wait()

    def zbody(j, carry):
        stripe_v[pl.ds(j * 16, 16)] = zeros16
        return carry

    lax.fori_loop(0, ROWS_PER_TILE // 16, zbody, 0)

    pltpu.sync_copy(stripe_v, deg_acc.at[pl.ds(s * ROWS_PER_TILE, ROWS_PER_TILE)])
    plsc.subcore_barrier()

    def body(i, carry):
        def ub(j, cc):
            v = pk_v[i, pl.ds(j * 16, 16)]
            idx_d[pl.ds(j * 16, 16)] = v & 16383
            return cc

        lax.fori_loop(0, CHUNK // 16, ub, 0)
        pltpu.sync_copy(ones_v, deg_acc.at[idx_d], add=True)
        return carry

    lax.fori_loop(0, RPT, body, 0)

    plsc.subcore_barrier()
    pltpu.sync_copy(deg_acc.at[pl.ds(s * ROWS_PER_TILE, ROWS_PER_TILE)], stripe_v)
    pltpu.sync_copy(
        stripe_v,
        deg_out.at[pl.ds(c * N_PAD + s * ROWS_PER_TILE, ROWS_PER_TILE)],
    )


def _deg_kernel(packed3):
    fn = pl.kernel(
        _deg_body,
        out_type=jax.ShapeDtypeStruct((NC * N_PAD,), jnp.float32),
        mesh=_sc_mesh(),
        scratch_types=[
            pltpu.VMEM((RPT, CHUNK), jnp.int32),
            pltpu.VMEM((CHUNK,), jnp.int32),
            pltpu.VMEM((CHUNK,), jnp.float32),
            pltpu.VMEM((ROWS_PER_TILE,), jnp.float32),
            pltpu.VMEM_SHARED((N_PAD,), jnp.float32),
            pltpu.SemaphoreType.DMA,
        ],
    )
    return fn(packed3)


def _hop_body(packed2, x, out, pk_v, src_c, dst_c, rows, acc, isem, gsem):
    c = lax.axis_index("c")
    s = lax.axis_index("s")

    base = jnp.where(c == 0, s * RPT0, NS * RPT0 + s * RPT1)
    nch = jnp.where(c == 0, RPT0, RPT1)

    zeros16 = jnp.zeros((16,), jnp.float32)

    idx_cp = pltpu.async_copy(packed2.at[pl.ds(base, RPT_MAX)], pk_v, isem)

    def zbody(k, carry):
        rows[0, k // 8, pl.ds((k % 8) * 16, 16)] = zeros16
        return carry

    lax.fori_loop(0, CHUNK * 8, zbody, 0)

    def zcopy(t, carry):
        pltpu.sync_copy(
            rows.at[0], acc.at[pl.ds(s * ROWS_PER_TILE + t * CHUNK, CHUNK)]
        )
        return carry

    lax.fori_loop(0, ROWS_PER_TILE // CHUNK, zcopy, 0)
    idx_cp.wait()
    plsc.subcore_barrier()

    def unpack(g, b):
        def ub(j, cc):
            v = pk_v[g, pl.ds(j * 16, 16)]
            src_c[b, pl.ds(j * 16, 16)] = lax.shift_right_logical(v, 14)
            dst_c[b, pl.ds(j * 16, 16)] = v & 16383
            return cc

        lax.fori_loop(0, CHUNK // 16, ub, 0)

    def start_gather(b):
        return pltpu.async_copy(x.at[src_c.at[b]], rows.at[b], gsem.at[b])

    def drain(b):
        pltpu.make_async_copy(x.at[src_c.at[b]], rows.at[b], gsem.at[b]).wait()

    for b in range(NBUF):
        unpack(b, b)
        start_gather(b)

    def body(t, carry):
        for b in range(NBUF):
            g = t * NBUF + b
            drain(b)
            pltpu.sync_copy(rows.at[b], acc.at[dst_c.at[b]], add=True)
            unpack(g + NBUF, b)
            start_gather(b)
        return carry

    lax.fori_loop(0, nch // NBUF - 1, body, 0)

    for b in range(NBUF):
        drain(b)
        pltpu.sync_copy(rows.at[b], acc.at[dst_c.at[b]], add=True)

    plsc.subcore_barrier()

    def ocopy(t, carry):
        r0 = s * ROWS_PER_TILE + t * CHUNK
        pltpu.sync_copy(acc.at[pl.ds(r0, CHUNK)], rows.at[0])
        pltpu.sync_copy(rows.at[0], out.at[c, pl.ds(r0, CHUNK)])
        return carry

    lax.fori_loop(0, ROWS_PER_TILE // CHUNK, ocopy, 0)


def _hop_kernel(packed2, x):
    fn = pl.kernel(
        _hop_body,
        out_type=jax.ShapeDtypeStruct((NC, N_PAD, F), jnp.float32),
        mesh=_sc_mesh(),
        scratch_types=[
            pltpu.VMEM((RPT_MAX, CHUNK), jnp.int32),
            pltpu.VMEM((NBUF, CHUNK), jnp.int32),
            pltpu.VMEM((NBUF, CHUNK), jnp.int32),
            pltpu.VMEM((NBUF, CHUNK, F), jnp.float32),
            pltpu.VMEM_SHARED((N_PAD, F), jnp.float32),
            pltpu.SemaphoreType.DMA,
            pltpu.SemaphoreType.DMA((NBUF,)),
        ],
    )
    return fn(packed2, x)


NB = 2000
GRID = N // NB


def _scale_body(feat_ref, norm_ref, x1_ref):
    x1_ref[...] = feat_ref[...] * norm_ref[...]


def _scale_kernel(feat, norm):
    return pl.pallas_call(
        _scale_body,
        grid=(GRID,),
        in_specs=[
            pl.BlockSpec((NB, F), lambda i: (i, 0)),
            pl.BlockSpec((NB, 1), lambda i: (i, 0)),
        ],
        out_specs=pl.BlockSpec((NB, F), lambda i: (i, 0)),
        out_shape=jax.ShapeDtypeStruct((N, F), jnp.float32),
    )(feat, norm)


def _make_combine_body(want_next):
    def body(parts_ref, norm_ref, *outs):
        p = parts_ref[0] + parts_ref[1]
        norm = norm_ref[...]
        h = p * norm
        outs[0][...] = h
        if want_next:
            outs[1][...] = h * norm
    return body


def _combine_kernel(parts, norm, want_next):
    shapes = [jax.ShapeDtypeStruct((N, F), jnp.float32)]
    specs = [pl.BlockSpec((NB, F), lambda i: (i, 0))]
    if want_next:
        shapes.append(jax.ShapeDtypeStruct((N, F), jnp.float32))
        specs.append(pl.BlockSpec((NB, F), lambda i: (i, 0)))
    return pl.pallas_call(
        _make_combine_body(want_next),
        grid=(GRID,),
        in_specs=[
            pl.BlockSpec((NC, NB, F), lambda i: (0, i, 0)),
            pl.BlockSpec((NB, 1), lambda i: (i, 0)),
        ],
        out_specs=tuple(specs),
        out_shape=tuple(shapes),
    )(parts, norm)


def _final_body(feat_ref, h1_ref, h2_ref, lam_w_ref, lam_b_ref, alpha_ref,
                fc_wt_ref, fc_b_ref, res_ref, ent_ref):
    xs = (feat_ref[...], h1_ref[...], h2_ref[...])
    lam_w = lam_w_ref[...]
    logits = [[jnp.sum(xs[k] * lam_w[h][None, :], axis=1, keepdims=True)
               + lam_b_ref[0, h]
               for h in range(NH)] for k in range(K + 1)]
    g = [jnp.zeros((NB, 1), jnp.float32) for _ in range(K + 1)]
    ent = jnp.zeros((NB, 1), jnp.float32)
    for h in range(NH):
        lk = [logits[k][h] for k in range(K + 1)]
        m = jnp.maximum(jnp.maximum(lk[0], lk[1]), lk[2])
        e = [jnp.exp(l - m) for l in lk]
        z = e[0] + e[1] + e[2]
        inv_z = 1.0 / z
        for k in range(K + 1):
            p = e[k] * inv_z
            g[k] = g[k] + p
            ent = ent - p * jnp.log(p + 1e-12)
    combined = jnp.zeros((NB, F), jnp.float32)
    for k in range(K + 1):
        combined = combined + xs[k] * (alpha_ref[0, k] * g[k])
    res = jnp.dot(combined, fc_wt_ref[...],
                  preferred_element_type=jnp.float32)
    res_ref[...] = res + 3.0 * fc_b_ref[...]
    ent_ref[...] = ent


def _final_kernel(feat, h1, h2, lam_w, lam_b, alpha, fc_wt, fc_b):
    row_spec = pl.BlockSpec((NB, F), lambda i: (i, 0))
    return pl.pallas_call(
        _final_body,
        grid=(GRID,),
        in_specs=[
            row_spec, row_spec, row_spec,
            pl.BlockSpec((NH, F), lambda i: (0, 0)),
            pl.BlockSpec((1, NH), lambda i: (0, 0)),
            pl.BlockSpec((1, K + 1), lambda i: (0, 0)),
            pl.BlockSpec((F, F), lambda i: (0, 0)),
            pl.BlockSpec((1, F), lambda i: (0, 0)),
        ],
        out_specs=(
            row_spec,
            pl.BlockSpec((NB, 1), lambda i: (i, 0)),
        ),
        out_shape=(
            jax.ShapeDtypeStruct((N, F), jnp.float32),
            jax.ShapeDtypeStruct((N, 1), jnp.float32),
        ),
    )(feat, h1, h2, lam_w, lam_b, alpha, fc_wt, fc_b)


def kernel(feat, edge_index, fc_w, fc_b, alpha, lam_w, lam_b):
    src = edge_index[0]
    dst = edge_index[1]
    pad = E_PAD - E
    src_p = jnp.concatenate([src, jnp.zeros((pad,), jnp.int32)])
    dst_p = jnp.concatenate([dst, jnp.full((pad,), N, jnp.int32)])
    packed = src_p * 16384 + dst_p
    packed3 = packed.reshape(NW, RPT, CHUNK)
    packed2 = jnp.concatenate(
        [packed.reshape(R_PAD, CHUNK),
         jnp.full((RPT_MAX, CHUNK), N, jnp.int32)])
    deg_parts = _deg_kernel(packed3).reshape(NC, N_PAD)
    deg = (deg_parts[0, :N] + deg_parts[1, :N])
    norm = lax.rsqrt(jnp.maximum(deg, 1.0))[:, None]
    x1 = _scale_kernel(feat, norm)

    parts1 = _hop_kernel(packed2, x1)
    h1, x2 = _combine_kernel(parts1, norm, want_next=True)

    parts2 = _hop_kernel(packed2, x2)
    (h2,) = _combine_kernel(parts2, norm, want_next=False)

    res, ent = _final_kernel(
        feat, h1, h2, lam_w,
        lam_b.reshape(1, NH), alpha.reshape(1, K + 1),
        fc_w.T, fc_b.reshape(1, F),
    )
    return (res, ent.reshape(N))


# speedup vs baseline: 3.0006x; 3.0006x over previous
import functools

import jax
import jax.numpy as jnp
from jax import lax
from jax.experimental import pallas as pl
from jax.experimental.pallas import tpu as pltpu
from jax.experimental.pallas import tpu_sc as plsc

N = 10000
E = 320000
F = 128
K = 2
NH = 3

NC = 2
NS = 16
NW = NC * NS

CHUNK = 128
RPT = 80
R_PAD = RPT * NW
E_PAD = R_PAD * CHUNK
NBUF = 2
RPT0 = 80
RPT1 = 80
RPT_MAX = max(RPT0, RPT1)
N_PAD = 10240
ROWS_PER_TILE = N_PAD // NS


def _sc_mesh():
    return plsc.VectorSubcoreMesh(core_axis_name="c", subcore_axis_name="s")


def _deg_body(packed3, deg_out, pk_v, idx_d, ones_v, stripe_v, deg_acc, sem):
    c = lax.axis_index("c")
    s = lax.axis_index("s")
    wid = c * NS + s

    ones16 = jnp.ones((16,), jnp.float32)
    zeros16 = jnp.zeros((16,), jnp.float32)
    for j in range(CHUNK // 16):
        ones_v[pl.ds(j * 16, 16)] = ones16

    pltpu.async_copy(packed3.at[wid], pk_v, sem).wait()

    def zbody(j, carry):
        stripe_v[pl.ds(j * 16, 16)] = zeros16
        return carry

    lax.fori_loop(0, ROWS_PER_TILE // 16, zbody, 0)

    pltpu.sync_copy(stripe_v, deg_acc.at[pl.ds(s * ROWS_PER_TILE, ROWS_PER_TILE)])
    plsc.subcore_barrier()

    def body(i, carry):
        def ub(j, cc):
            v = pk_v[i, pl.ds(j * 16, 16)]
            idx_d[pl.ds(j * 16, 16)] = v & 16383
            return cc

        lax.fori_loop(0, CHUNK // 16, ub, 0)
        pltpu.sync_copy(ones_v, deg_acc.at[idx_d], add=True)
        return carry

    lax.fori_loop(0, RPT, body, 0)

    plsc.subcore_barrier()
    pltpu.sync_copy(deg_acc.at[pl.ds(s * ROWS_PER_TILE, ROWS_PER_TILE)], stripe_v)
    pltpu.sync_copy(
        stripe_v,
        deg_out.at[pl.ds(c * N_PAD + s * ROWS_PER_TILE, ROWS_PER_TILE)],
    )


def _deg_kernel(packed3):
    fn = pl.kernel(
        _deg_body,
        out_type=jax.ShapeDtypeStruct((NC * N_PAD,), jnp.float32),
        mesh=_sc_mesh(),
        scratch_types=[
            pltpu.VMEM((RPT, CHUNK), jnp.int32),
            pltpu.VMEM((CHUNK,), jnp.int32),
            pltpu.VMEM((CHUNK,), jnp.float32),
            pltpu.VMEM((ROWS_PER_TILE,), jnp.float32),
            pltpu.VMEM_SHARED((N_PAD,), jnp.float32),
            pltpu.SemaphoreType.DMA,
        ],
    )
    return fn(packed3)


def _hop_body(packed2, x, out, pk_v, src_c, dst_c, rows, acc, isem, gsem):
    c = lax.axis_index("c")
    s = lax.axis_index("s")

    base = jnp.where(c == 0, s * RPT0, NS * RPT0 + s * RPT1)
    nch = jnp.where(c == 0, RPT0, RPT1)

    zeros16 = jnp.zeros((16,), jnp.float32)

    idx_cp = pltpu.async_copy(packed2.at[pl.ds(base, RPT_MAX)], pk_v, isem)

    def zbody(k, carry):
        rows[0, k // 8, pl.ds((k % 8) * 16, 16)] = zeros16
        return carry

    lax.fori_loop(0, CHUNK * 8, zbody, 0)

    def zcopy(t, carry):
        pltpu.sync_copy(
            rows.at[0], acc.at[pl.ds(s * ROWS_PER_TILE + t * CHUNK, CHUNK)]
        )
        return carry

    lax.fori_loop(0, ROWS_PER_TILE // CHUNK, zcopy, 0)
    idx_cp.wait()
    plsc.subcore_barrier()

    def unpack(g, b):
        def ub(j, cc):
            v = pk_v[g, pl.ds(j * 16, 16)]
            src_c[b, pl.ds(j * 16, 16)] = lax.shift_right_logical(v, 14)
            dst_c[b, pl.ds(j * 16, 16)] = v & 16383
            return cc

        lax.fori_loop(0, CHUNK // 16, ub, 0)

    def start_gather(b):
        return pltpu.async_copy(x.at[src_c.at[b]], rows.at[b], gsem.at[b])

    def drain(b):
        pltpu.make_async_copy(x.at[src_c.at[b]], rows.at[b], gsem.at[b]).wait()

    for b in range(NBUF):
        unpack(b, b)
        start_gather(b)

    def body(t, carry):
        for b in range(NBUF):
            g = t * NBUF + b
            drain(b)
            pltpu.sync_copy(rows.at[b], acc.at[dst_c.at[b]], add=True)
            unpack(g + NBUF, b)
            start_gather(b)
        return carry

    lax.fori_loop(0, nch // NBUF - 1, body, 0)

    for b in range(NBUF):
        drain(b)
        pltpu.sync_copy(rows.at[b], acc.at[dst_c.at[b]], add=True)

    plsc.subcore_barrier()

    def ocopy(t, carry):
        r0 = s * ROWS_PER_TILE + t * CHUNK
        pltpu.sync_copy(acc.at[pl.ds(r0, CHUNK)], rows.at[0])
        pltpu.sync_copy(rows.at[0], out.at[c, pl.ds(r0, CHUNK)])
        return carry

    lax.fori_loop(0, ROWS_PER_TILE // CHUNK, ocopy, 0)


def _hop_kernel(packed2, x):
    fn = pl.kernel(
        _hop_body,
        out_type=jax.ShapeDtypeStruct((NC, N_PAD, F), jnp.float32),
        mesh=_sc_mesh(),
        scratch_types=[
            pltpu.VMEM((RPT_MAX, CHUNK), jnp.int32),
            pltpu.VMEM((NBUF, CHUNK), jnp.int32),
            pltpu.VMEM((NBUF, CHUNK), jnp.int32),
            pltpu.VMEM((NBUF, CHUNK, F), jnp.float32),
            pltpu.VMEM_SHARED((N_PAD, F), jnp.float32),
            pltpu.SemaphoreType.DMA,
            pltpu.SemaphoreType.DMA((NBUF,)),
        ],
    )
    return fn(packed2, x)


NB = 2000
GRID = N // NB


def _scale_body(feat_ref, norm_ref, x1_ref):
    x1_ref[...] = feat_ref[...] * norm_ref[...]


def _scale_kernel(feat, norm):
    return pl.pallas_call(
        _scale_body,
        grid=(GRID,),
        in_specs=[
            pl.BlockSpec((NB, F), lambda i: (i, 0)),
            pl.BlockSpec((NB, 1), lambda i: (i, 0)),
        ],
        out_specs=pl.BlockSpec((NB, F), lambda i: (i, 0)),
        out_shape=jax.ShapeDtypeStruct((N, F), jnp.float32),
    )(feat, norm)


def _make_combine_body(want_next):
    def body(parts_ref, norm_ref, *outs):
        p = parts_ref[0] + parts_ref[1]
        norm = norm_ref[...]
        h = p * norm
        outs[0][...] = h
        if want_next:
            outs[1][...] = h * norm
    return body


def _combine_kernel(parts, norm, want_next):
    shapes = [jax.ShapeDtypeStruct((N, F), jnp.float32)]
    specs = [pl.BlockSpec((NB, F), lambda i: (i, 0))]
    if want_next:
        shapes.append(jax.ShapeDtypeStruct((N, F), jnp.float32))
        specs.append(pl.BlockSpec((NB, F), lambda i: (i, 0)))
    return pl.pallas_call(
        _make_combine_body(want_next),
        grid=(GRID,),
        in_specs=[
            pl.BlockSpec((NC, NB, F), lambda i: (0, i, 0)),
            pl.BlockSpec((NB, 1), lambda i: (i, 0)),
        ],
        out_specs=tuple(specs),
        out_shape=tuple(shapes),
    )(parts, norm)


def _final_body(feat_ref, h1_ref, h2_ref, lam_w_ref, lam_b_ref, alpha_ref,
                fc_wt_ref, fc_b_ref, res_ref, ent_ref):
    xs = (feat_ref[...], h1_ref[...], h2_ref[...])
    lam_w = lam_w_ref[...]
    logits = [[jnp.sum(xs[k] * lam_w[h][None, :], axis=1, keepdims=True)
               + lam_b_ref[0, h]
               for h in range(NH)] for k in range(K + 1)]
    g = [jnp.zeros((NB, 1), jnp.float32) for _ in range(K + 1)]
    ent = jnp.zeros((NB, 1), jnp.float32)
    for h in range(NH):
        lk = [logits[k][h] for k in range(K + 1)]
        m = jnp.maximum(jnp.maximum(lk[0], lk[1]), lk[2])
        e = [jnp.exp(l - m) for l in lk]
        z = e[0] + e[1] + e[2]
        inv_z = 1.0 / z
        for k in range(K + 1):
            p = e[k] * inv_z
            g[k] = g[k] + p
            ent = ent - p * jnp.log(p + 1e-12)
    combined = jnp.zeros((NB, F), jnp.float32)
    for k in range(K + 1):
        combined = combined + xs[k] * (alpha_ref[0, k] * g[k])
    res = jnp.dot(combined, fc_wt_ref[...],
                  preferred_element_type=jnp.float32)
    res_ref[...] = res + 3.0 * fc_b_ref[...]
    ent_ref[...] = ent


def _final_kernel(feat, h1, h2, lam_w, lam_b, alpha, fc_wt, fc_b):
    row_spec = pl.BlockSpec((NB, F), lambda i: (i, 0))
    return pl.pallas_call(
        _final_body,
        grid=(GRID,),
        in_specs=[
            row_spec, row_spec, row_spec,
            pl.BlockSpec((NH, F), lambda i: (0, 0)),
            pl.BlockSpec((1, NH), lambda i: (0, 0)),
            pl.BlockSpec((1, K + 1), lambda i: (0, 0)),
            pl.BlockSpec((F, F), lambda i: (0, 0)),
            pl.BlockSpec((1, F), lambda i: (0, 0)),
        ],
        out_specs=(
            row_spec,
            pl.BlockSpec((NB, 1), lambda i: (i, 0)),
        ),
        out_shape=(
            jax.ShapeDtypeStruct((N, F), jnp.float32),
            jax.ShapeDtypeStruct((N, 1), jnp.float32),
        ),
    )(feat, h1, h2, lam_w, lam_b, alpha, fc_wt, fc_b)


def kernel(feat, edge_index, fc_w, fc_b, alpha, lam_w, lam_b):
    src = edge_index[0]
    dst = edge_index[1]
    pad = E_PAD - E
    dummy_i = jnp.arange(pad, dtype=jnp.int32)
    src_p = jnp.concatenate([src, dummy_i % N])
    dst_p = jnp.concatenate([dst, N + (dummy_i % (N_PAD - N))])
    packed = src_p * 16384 + dst_p
    packed3 = packed.reshape(NW, RPT, CHUNK)
    packed2 = jnp.concatenate(
        [packed.reshape(R_PAD, CHUNK),
         jnp.full((RPT_MAX, CHUNK), N, jnp.int32)])
    deg_parts = _deg_kernel(packed3).reshape(NC, N_PAD)
    deg = (deg_parts[0, :N] + deg_parts[1, :N])
    norm = lax.rsqrt(jnp.maximum(deg, 1.0))[:, None]
    x1 = _scale_kernel(feat, norm)

    parts1 = _hop_kernel(packed2, x1)
    h1, x2 = _combine_kernel(parts1, norm, want_next=True)

    parts2 = _hop_kernel(packed2, x2)
    (h2,) = _combine_kernel(parts2, norm, want_next=False)

    res, ent = _final_kernel(
        feat, h1, h2, lam_w,
        lam_b.reshape(1, NH), alpha.reshape(1, K + 1),
        fc_w.T, fc_b.reshape(1, F),
    )
    return (res, ent.reshape(N))


# speedup vs baseline: 3.0585x; 1.0193x over previous
import functools

import jax
import jax.numpy as jnp
from jax import lax
from jax.experimental import pallas as pl
from jax.experimental.pallas import tpu as pltpu
from jax.experimental.pallas import tpu_sc as plsc

N = 10000
E = 320000
F = 128
K = 2
NH = 3

NC = 2
NS = 16
NW = NC * NS

CHUNK = 128
RPT = 80
R_PAD = RPT * NW
E_PAD = R_PAD * CHUNK
NBUF = 2
RPT0 = 80
RPT1 = 80
RPT_MAX = max(RPT0, RPT1)
N_PAD = 10240
ROWS_PER_TILE = N_PAD // NS


def _sc_mesh():
    return plsc.VectorSubcoreMesh(core_axis_name="c", subcore_axis_name="s")


def _deg_body(packed3, deg_out, pk_v, idx_d, ones_v, stripe_v, deg_acc, sem):
    c = lax.axis_index("c")
    s = lax.axis_index("s")
    wid = c * NS + s

    ones16 = jnp.ones((16,), jnp.float32)
    zeros16 = jnp.zeros((16,), jnp.float32)
    for j in range(CHUNK // 16):
        ones_v[pl.ds(j * 16, 16)] = ones16

    pltpu.async_copy(packed3.at[wid], pk_v, sem).wait()

    def zbody(j, carry):
        stripe_v[pl.ds(j * 16, 16)] = zeros16
        return carry

    lax.fori_loop(0, ROWS_PER_TILE // 16, zbody, 0)

    pltpu.sync_copy(stripe_v, deg_acc.at[pl.ds(s * ROWS_PER_TILE, ROWS_PER_TILE)])
    plsc.subcore_barrier()

    def body(i, carry):
        def ub(j, cc):
            v = pk_v[i, pl.ds(j * 16, 16)]
            idx_d[pl.ds(j * 16, 16)] = v & 16383
            return cc

        lax.fori_loop(0, CHUNK // 16, ub, 0)
        pltpu.sync_copy(ones_v, deg_acc.at[idx_d], add=True)
        return carry

    lax.fori_loop(0, RPT, body, 0)

    plsc.subcore_barrier()
    pltpu.sync_copy(deg_acc.at[pl.ds(s * ROWS_PER_TILE, ROWS_PER_TILE)], stripe_v)
    pltpu.sync_copy(
        stripe_v,
        deg_out.at[pl.ds(c * N_PAD + s * ROWS_PER_TILE, ROWS_PER_TILE)],
    )


def _deg_kernel(packed3):
    fn = pl.kernel(
        _deg_body,
        out_type=jax.ShapeDtypeStruct((NC * N_PAD,), jnp.float32),
        mesh=_sc_mesh(),
        scratch_types=[
            pltpu.VMEM((RPT, CHUNK), jnp.int32),
            pltpu.VMEM((CHUNK,), jnp.int32),
            pltpu.VMEM((CHUNK,), jnp.float32),
            pltpu.VMEM((ROWS_PER_TILE,), jnp.float32),
            pltpu.VMEM_SHARED((N_PAD,), jnp.float32),
            pltpu.SemaphoreType.DMA,
        ],
    )
    return fn(packed3)


def _hop_body(packed2, x, out, pk_v, src_c, dst_c, rows, acc, isem, gsem):
    c = lax.axis_index("c")
    s = lax.axis_index("s")

    base = jnp.where(c == 0, s * RPT0, NS * RPT0 + s * RPT1)
    nch = jnp.where(c == 0, RPT0, RPT1)

    zeros16 = jnp.zeros((16,), jnp.float32)

    idx_cp = pltpu.async_copy(packed2.at[pl.ds(base, RPT_MAX)], pk_v, isem)

    def zbody(k, carry):
        rows[0, k // 8, pl.ds((k % 8) * 16, 16)] = zeros16
        return carry

    lax.fori_loop(0, CHUNK * 8, zbody, 0)

    def zcopy(t, carry):
        pltpu.sync_copy(
            rows.at[0], acc.at[pl.ds(s * ROWS_PER_TILE + t * CHUNK, CHUNK)]
        )
        return carry

    lax.fori_loop(0, ROWS_PER_TILE // CHUNK, zcopy, 0)
    idx_cp.wait()
    plsc.subcore_barrier()

    def unpack(g, b):
        def ub(j, cc):
            v = pk_v[g, pl.ds(j * 16, 16)]
            src_c[b, pl.ds(j * 16, 16)] = lax.shift_right_logical(v, 14)
            dst_c[b, pl.ds(j * 16, 16)] = v & 16383
            return cc

        lax.fori_loop(0, CHUNK // 16, ub, 0)

    def start_gather(b):
        return pltpu.async_copy(x.at[src_c.at[b]], rows.at[b], gsem.at[b])

    def drain(b):
        pltpu.make_async_copy(x.at[src_c.at[b]], rows.at[b], gsem.at[b]).wait()

    for b in range(NBUF):
        unpack(b, b)
        start_gather(b)

    def body(t, carry):
        for b in range(NBUF):
            g = t * NBUF + b
            drain(b)
            pltpu.sync_copy(rows.at[b], acc.at[dst_c.at[b]], add=True)
            unpack(g + NBUF, b)
            start_gather(b)
        return carry

    lax.fori_loop(0, nch // NBUF - 1, body, 0)

    for b in range(NBUF):
        drain(b)
        pltpu.sync_copy(rows.at[b], acc.at[dst_c.at[b]], add=True)

    plsc.subcore_barrier()

    def ocopy(t, carry):
        r0 = s * ROWS_PER_TILE + t * CHUNK
        pltpu.sync_copy(acc.at[pl.ds(r0, CHUNK)], rows.at[0])
        pltpu.sync_copy(rows.at[0], out.at[c, pl.ds(r0, CHUNK)])
        return carry

    lax.fori_loop(0, ROWS_PER_TILE // CHUNK, ocopy, 0)


def _hop_kernel(packed2, x):
    fn = pl.kernel(
        _hop_body,
        out_type=jax.ShapeDtypeStruct((NC, N_PAD, F), jnp.float32),
        mesh=_sc_mesh(),
        scratch_types=[
            pltpu.VMEM((RPT_MAX, CHUNK), jnp.int32),
            pltpu.VMEM((NBUF, CHUNK), jnp.int32),
            pltpu.VMEM((NBUF, CHUNK), jnp.int32),
            pltpu.VMEM((NBUF, CHUNK, F), jnp.float32),
            pltpu.VMEM_SHARED((N_PAD, F), jnp.float32),
            pltpu.SemaphoreType.DMA,
            pltpu.SemaphoreType.DMA((NBUF,)),
        ],
    )
    return fn(packed2, x)


NB = 2000
GRID = N // NB


def _scale_body(feat_ref, norm_ref, x1_ref):
    x1_ref[...] = feat_ref[...] * norm_ref[...]


def _scale_kernel(feat, norm):
    return pl.pallas_call(
        _scale_body,
        grid=(GRID,),
        in_specs=[
            pl.BlockSpec((NB, F), lambda i: (i, 0)),
            pl.BlockSpec((NB, 1), lambda i: (i, 0)),
        ],
        out_specs=pl.BlockSpec((NB, F), lambda i: (i, 0)),
        out_shape=jax.ShapeDtypeStruct((N, F), jnp.float32),
    )(feat, norm)


def _combine_body(parts_ref, norm_ref, x2_ref):
    p = parts_ref[0] + parts_ref[1]
    norm = norm_ref[...]
    x2_ref[...] = p * norm * norm


def _combine_kernel(parts, norm):
    return pl.pallas_call(
        _combine_body,
        grid=(GRID,),
        in_specs=[
            pl.BlockSpec((NC, NB, F), lambda i: (0, i, 0)),
            pl.BlockSpec((NB, 1), lambda i: (i, 0)),
        ],
        out_specs=pl.BlockSpec((NB, F), lambda i: (i, 0)),
        out_shape=jax.ShapeDtypeStruct((N, F), jnp.float32),
    )(parts, norm)


def _final_body(feat_ref, parts1_ref, parts2_ref, norm_ref, lam_w_ref,
                lam_b_ref, alpha_ref, fc_wt_ref, fc_b_ref, res_ref, ent_ref):
    norm = norm_ref[...]
    h1 = (parts1_ref[0] + parts1_ref[1]) * norm
    h2 = (parts2_ref[0] + parts2_ref[1]) * norm
    xs = (feat_ref[...], h1, h2)
    lam_w = lam_w_ref[...]
    logits = [[jnp.sum(xs[k] * lam_w[h][None, :], axis=1, keepdims=True)
               + lam_b_ref[0, h]
               for h in range(NH)] for k in range(K + 1)]
    g = [jnp.zeros((NB, 1), jnp.float32) for _ in range(K + 1)]
    ent = jnp.zeros((NB, 1), jnp.float32)
    for h in range(NH):
        lk = [logits[k][h] for k in range(K + 1)]
        m = jnp.maximum(jnp.maximum(lk[0], lk[1]), lk[2])
        e = [jnp.exp(l - m) for l in lk]
        z = e[0] + e[1] + e[2]
        inv_z = 1.0 / z
        for k in range(K + 1):
            p = e[k] * inv_z
            g[k] = g[k] + p
            ent = ent - p * jnp.log(p + 1e-12)
    combined = jnp.zeros((NB, F), jnp.float32)
    for k in range(K + 1):
        combined = combined + xs[k] * (alpha_ref[0, k] * g[k])
    res = jnp.dot(combined, fc_wt_ref[...],
                  preferred_element_type=jnp.float32)
    res_ref[...] = res + 3.0 * fc_b_ref[...]
    ent_ref[...] = ent


def _final_kernel(feat, parts1, parts2, norm, lam_w, lam_b, alpha, fc_wt, fc_b):
    row_spec = pl.BlockSpec((NB, F), lambda i: (i, 0))
    parts_spec = pl.BlockSpec((NC, NB, F), lambda i: (0, i, 0))
    return pl.pallas_call(
        _final_body,
        grid=(GRID,),
        in_specs=[
            row_spec, parts_spec, parts_spec,
            pl.BlockSpec((NB, 1), lambda i: (i, 0)),
            pl.BlockSpec((NH, F), lambda i: (0, 0)),
            pl.BlockSpec((1, NH), lambda i: (0, 0)),
            pl.BlockSpec((1, K + 1), lambda i: (0, 0)),
            pl.BlockSpec((F, F), lambda i: (0, 0)),
            pl.BlockSpec((1, F), lambda i: (0, 0)),
        ],
        out_specs=(
            row_spec,
            pl.BlockSpec((NB, 1), lambda i: (i, 0)),
        ),
        out_shape=(
            jax.ShapeDtypeStruct((N, F), jnp.float32),
            jax.ShapeDtypeStruct((N, 1), jnp.float32),
        ),
    )(feat, parts1, parts2, norm, lam_w, lam_b, alpha, fc_wt, fc_b)


def kernel(feat, edge_index, fc_w, fc_b, alpha, lam_w, lam_b):
    src = edge_index[0]
    dst = edge_index[1]
    pad = E_PAD - E
    dummy_i = jnp.arange(pad, dtype=jnp.int32)
    src_p = jnp.concatenate([src, dummy_i % N])
    dst_p = jnp.concatenate([dst, N + (dummy_i % (N_PAD - N))])
    packed = src_p * 16384 + dst_p
    packed3 = packed.reshape(NW, RPT, CHUNK)
    packed2 = jnp.concatenate(
        [packed.reshape(R_PAD, CHUNK),
         jnp.full((RPT_MAX, CHUNK), N, jnp.int32)])
    deg_parts = _deg_kernel(packed3).reshape(NC, N_PAD)
    deg = (deg_parts[0, :N] + deg_parts[1, :N])
    norm = lax.rsqrt(jnp.maximum(deg, 1.0))[:, None]
    x1 = _scale_kernel(feat, norm)

    parts1 = _hop_kernel(packed2, x1)
    x2 = _combine_kernel(parts1, norm)

    parts2 = _hop_kernel(packed2, x2)

    res, ent = _final_kernel(
        feat, parts1, parts2, norm, lam_w,
        lam_b.reshape(1, NH), alpha.reshape(1, K + 1),
        fc_w.T, fc_b.reshape(1, F),
    )
    return (res, ent.reshape(N))


# speedup vs baseline: 3.0625x; 1.0013x over previous
import functools

import jax
import jax.numpy as jnp
from jax import lax
from jax.experimental import pallas as pl
from jax.experimental.pallas import tpu as pltpu
from jax.experimental.pallas import tpu_sc as plsc

N = 10000
E = 320000
F = 128
K = 2
NH = 3

NC = 2
NS = 16
NW = NC * NS

CHUNK = 128
RPT = 80
R_PAD = RPT * NW
E_PAD = R_PAD * CHUNK
NBUF = 2
RPT0 = 80
RPT1 = 80
RPT_MAX = max(RPT0, RPT1)
N_PAD = 10240
ROWS_PER_TILE = N_PAD // NS


def _sc_mesh():
    return plsc.VectorSubcoreMesh(core_axis_name="c", subcore_axis_name="s")


def _deg_body(packed3, deg_out, pk_v, idx_d, ones_v, stripe_v, deg_acc, sem):
    c = lax.axis_index("c")
    s = lax.axis_index("s")
    wid = c * NS + s

    ones16 = jnp.ones((16,), jnp.float32)
    zeros16 = jnp.zeros((16,), jnp.float32)
    for j in range(CHUNK // 16):
        ones_v[pl.ds(j * 16, 16)] = ones16

    pltpu.async_copy(packed3.at[wid], pk_v, sem).wait()

    def zbody(j, carry):
        stripe_v[pl.ds(j * 16, 16)] = zeros16
        return carry

    lax.fori_loop(0, ROWS_PER_TILE // 16, zbody, 0)

    pltpu.sync_copy(stripe_v, deg_acc.at[pl.ds(s * ROWS_PER_TILE, ROWS_PER_TILE)])
    plsc.subcore_barrier()

    def body(i, carry):
        def ub(j, cc):
            v = pk_v[i, pl.ds(j * 16, 16)]
            idx_d[pl.ds(j * 16, 16)] = v & 16383
            return cc

        lax.fori_loop(0, CHUNK // 16, ub, 0)
        pltpu.sync_copy(ones_v, deg_acc.at[idx_d], add=True)
        return carry

    lax.fori_loop(0, RPT, body, 0)

    plsc.subcore_barrier()
    pltpu.sync_copy(deg_acc.at[pl.ds(s * ROWS_PER_TILE, ROWS_PER_TILE)], stripe_v)
    pltpu.sync_copy(
        stripe_v,
        deg_out.at[pl.ds(c * N_PAD + s * ROWS_PER_TILE, ROWS_PER_TILE)],
    )


def _deg_kernel(packed3):
    fn = pl.kernel(
        _deg_body,
        out_type=jax.ShapeDtypeStruct((NC * N_PAD,), jnp.float32),
        mesh=_sc_mesh(),
        scratch_types=[
            pltpu.VMEM((RPT, CHUNK), jnp.int32),
            pltpu.VMEM((CHUNK,), jnp.int32),
            pltpu.VMEM((CHUNK,), jnp.float32),
            pltpu.VMEM((ROWS_PER_TILE,), jnp.float32),
            pltpu.VMEM_SHARED((N_PAD,), jnp.float32),
            pltpu.SemaphoreType.DMA,
        ],
    )
    return fn(packed3)


def _hop_body(packed2, x, out, pk_v, src_c, dst_c, rows, acc, isem, gsem):
    c = lax.axis_index("c")
    s = lax.axis_index("s")

    base = jnp.where(c == 0, s * RPT0, NS * RPT0 + s * RPT1)
    nch = jnp.where(c == 0, RPT0, RPT1)

    zeros16 = jnp.zeros((16,), jnp.float32)

    idx_cp = pltpu.async_copy(packed2.at[pl.ds(base, RPT_MAX)], pk_v, isem)

    def zbody(k, carry):
        rows[0, k // 8, pl.ds((k % 8) * 16, 16)] = zeros16
        return carry

    lax.fori_loop(0, CHUNK * 8, zbody, 0)

    def zcopy(t, carry):
        pltpu.sync_copy(
            rows.at[0], acc.at[pl.ds(s * ROWS_PER_TILE + t * CHUNK, CHUNK)]
        )
        return carry

    lax.fori_loop(0, ROWS_PER_TILE // CHUNK, zcopy, 0)
    idx_cp.wait()
    plsc.subcore_barrier()

    def unpack(g, b):
        def ub(j, cc):
            v = pk_v[g, pl.ds(j * 16, 16)]
            src_c[b, pl.ds(j * 16, 16)] = lax.shift_right_logical(v, 14)
            dst_c[b, pl.ds(j * 16, 16)] = v & 16383
            return cc

        lax.fori_loop(0, CHUNK // 16, ub, 0)

    H = CHUNK // 2

    def start_gather(b):
        pltpu.async_copy(
            x.at[src_c.at[b, pl.ds(0, H)]], rows.at[b, pl.ds(0, H)],
            gsem.at[b])
        pltpu.async_copy(
            x.at[src_c.at[b, pl.ds(H, H)]], rows.at[b, pl.ds(H, H)],
            gsem.at[b])

    def drain(b):
        pltpu.make_async_copy(
            x.at[src_c.at[b, pl.ds(0, H)]], rows.at[b, pl.ds(0, H)],
            gsem.at[b]).wait()
        pltpu.make_async_copy(
            x.at[src_c.at[b, pl.ds(H, H)]], rows.at[b, pl.ds(H, H)],
            gsem.at[b]).wait()

    for b in range(NBUF):
        unpack(b, b)
        start_gather(b)

    def body(t, carry):
        for b in range(NBUF):
            g = t * NBUF + b
            drain(b)
            pltpu.sync_copy(rows.at[b], acc.at[dst_c.at[b]], add=True)
            unpack(g + NBUF, b)
            start_gather(b)
        return carry

    lax.fori_loop(0, nch // NBUF - 1, body, 0)

    for b in range(NBUF):
        drain(b)
        pltpu.sync_copy(rows.at[b], acc.at[dst_c.at[b]], add=True)

    plsc.subcore_barrier()

    def ocopy(t, carry):
        r0 = s * ROWS_PER_TILE + t * CHUNK
        pltpu.sync_copy(acc.at[pl.ds(r0, CHUNK)], rows.at[0])
        pltpu.sync_copy(rows.at[0], out.at[c, pl.ds(r0, CHUNK)])
        return carry

    lax.fori_loop(0, ROWS_PER_TILE // CHUNK, ocopy, 0)


def _hop_kernel(packed2, x):
    fn = pl.kernel(
        _hop_body,
        out_type=jax.ShapeDtypeStruct((NC, N_PAD, F), jnp.float32),
        mesh=_sc_mesh(),
        scratch_types=[
            pltpu.VMEM((RPT_MAX, CHUNK), jnp.int32),
            pltpu.VMEM((NBUF, CHUNK), jnp.int32),
            pltpu.VMEM((NBUF, CHUNK), jnp.int32),
            pltpu.VMEM((NBUF, CHUNK, F), jnp.float32),
            pltpu.VMEM_SHARED((N_PAD, F), jnp.float32),
            pltpu.SemaphoreType.DMA,
            pltpu.SemaphoreType.DMA((NBUF,)),
        ],
    )
    return fn(packed2, x)


NB = 2000
GRID = N // NB


def _scale_body(feat_ref, norm_ref, x1_ref):
    x1_ref[...] = feat_ref[...] * norm_ref[...]


def _scale_kernel(feat, norm):
    return pl.pallas_call(
        _scale_body,
        grid=(GRID,),
        in_specs=[
            pl.BlockSpec((NB, F), lambda i: (i, 0)),
            pl.BlockSpec((NB, 1), lambda i: (i, 0)),
        ],
        out_specs=pl.BlockSpec((NB, F), lambda i: (i, 0)),
        out_shape=jax.ShapeDtypeStruct((N, F), jnp.float32),
    )(feat, norm)


def _combine_body(parts_ref, norm_ref, x2_ref):
    p = parts_ref[0] + parts_ref[1]
    norm = norm_ref[...]
    x2_ref[...] = p * norm * norm


def _combine_kernel(parts, norm):
    return pl.pallas_call(
        _combine_body,
        grid=(GRID,),
        in_specs=[
            pl.BlockSpec((NC, NB, F), lambda i: (0, i, 0)),
            pl.BlockSpec((NB, 1), lambda i: (i, 0)),
        ],
        out_specs=pl.BlockSpec((NB, F), lambda i: (i, 0)),
        out_shape=jax.ShapeDtypeStruct((N, F), jnp.float32),
    )(parts, norm)


def _final_body(feat_ref, parts1_ref, parts2_ref, norm_ref, lam_w_ref,
                lam_b_ref, alpha_ref, fc_wt_ref, fc_b_ref, res_ref, ent_ref):
    norm = norm_ref[...]
    h1 = (parts1_ref[0] + parts1_ref[1]) * norm
    h2 = (parts2_ref[0] + parts2_ref[1]) * norm
    xs = (feat_ref[...], h1, h2)
    lam_w = lam_w_ref[...]
    logits = [[jnp.sum(xs[k] * lam_w[h][None, :], axis=1, keepdims=True)
               + lam_b_ref[0, h]
               for h in range(NH)] for k in range(K + 1)]
    g = [jnp.zeros((NB, 1), jnp.float32) for _ in range(K + 1)]
    ent = jnp.zeros((NB, 1), jnp.float32)
    for h in range(NH):
        lk = [logits[k][h] for k in range(K + 1)]
        m = jnp.maximum(jnp.maximum(lk[0], lk[1]), lk[2])
        e = [jnp.exp(l - m) for l in lk]
        z = e[0] + e[1] + e[2]
        inv_z = 1.0 / z
        for k in range(K + 1):
            p = e[k] * inv_z
            g[k] = g[k] + p
            ent = ent - p * jnp.log(p + 1e-12)
    combined = jnp.zeros((NB, F), jnp.float32)
    for k in range(K + 1):
        combined = combined + xs[k] * (alpha_ref[0, k] * g[k])
    res = jnp.dot(combined, fc_wt_ref[...],
                  preferred_element_type=jnp.float32)
    res_ref[...] = res + 3.0 * fc_b_ref[...]
    ent_ref[...] = ent


def _final_kernel(feat, parts1, parts2, norm, lam_w, lam_b, alpha, fc_wt, fc_b):
    row_spec = pl.BlockSpec((NB, F), lambda i: (i, 0))
    parts_spec = pl.BlockSpec((NC, NB, F), lambda i: (0, i, 0))
    return pl.pallas_call(
        _final_body,
        grid=(GRID,),
        in_specs=[
            row_spec, parts_spec, parts_spec,
            pl.BlockSpec((NB, 1), lambda i: (i, 0)),
            pl.BlockSpec((NH, F), lambda i: (0, 0)),
            pl.BlockSpec((1, NH), lambda i: (0, 0)),
            pl.BlockSpec((1, K + 1), lambda i: (0, 0)),
            pl.BlockSpec((F, F), lambda i: (0, 0)),
            pl.BlockSpec((1, F), lambda i: (0, 0)),
        ],
        out_specs=(
            row_spec,
            pl.BlockSpec((NB, 1), lambda i: (i, 0)),
        ),
        out_shape=(
            jax.ShapeDtypeStruct((N, F), jnp.float32),
            jax.ShapeDtypeStruct((N, 1), jnp.float32),
        ),
    )(feat, parts1, parts2, norm, lam_w, lam_b, alpha, fc_wt, fc_b)


def kernel(feat, edge_index, fc_w, fc_b, alpha, lam_w, lam_b):
    src = edge_index[0]
    dst = edge_index[1]
    pad = E_PAD - E
    dummy_i = jnp.arange(pad, dtype=jnp.int32)
    src_p = jnp.concatenate([src, dummy_i % N])
    dst_p = jnp.concatenate([dst, N + (dummy_i % (N_PAD - N))])
    packed = src_p * 16384 + dst_p
    packed3 = packed.reshape(NW, RPT, CHUNK)
    packed2 = jnp.concatenate(
        [packed.reshape(R_PAD, CHUNK),
         jnp.full((RPT_MAX, CHUNK), N, jnp.int32)])
    deg_parts = _deg_kernel(packed3).reshape(NC, N_PAD)
    deg = (deg_parts[0, :N] + deg_parts[1, :N])
    norm = lax.rsqrt(jnp.maximum(deg, 1.0))[:, None]
    x1 = _scale_kernel(feat, norm)

    parts1 = _hop_kernel(packed2, x1)
    x2 = _combine_kernel(parts1, norm)

    parts2 = _hop_kernel(packed2, x2)

    res, ent = _final_kernel(
        feat, parts1, parts2, norm, lam_w,
        lam_b.reshape(1, NH), alpha.reshape(1, K + 1),
        fc_w.T, fc_b.reshape(1, F),
    )
    return (res, ent.reshape(N))
